# baseline (speedup 1.0000x reference)
import jax
import jax.numpy as jnp
from jax.experimental import pallas as pl

E = 64
TOP_K = 8
R = 16
D_IN = 768
D_OUT = 768
SCALING = 32.0 / R

TILE = 512

_DN_NT = (((1,), (1,)), ((), ()))


def _moe_lora_kernel(x_ref, ab3_ref, bb_ref, b2_ref, out_ref):
    xb = x_ref[:].astype(jnp.bfloat16)
    hb = jax.lax.dot_general(xb, ab3_ref[:], _DN_NT,
                             preferred_element_type=jnp.float32)
    h = hb[:, :E * R]
    base = hb[:, E * R:E * R + D_OUT]
    logits = hb[:, E * R + D_OUT:]
    p = jnp.exp(logits)

    colid = jax.lax.broadcasted_iota(jnp.int32, p.shape, 1)
    pbits = jax.lax.bitcast_convert_type(p, jnp.int32)
    keyi = jnp.bitwise_or(jnp.bitwise_and(pbits, jnp.int32(~63)),
                          (E - 1) - colid)
    key = jax.lax.bitcast_convert_type(keyi, jnp.float32)
    mask = jnp.zeros_like(p, dtype=jnp.bool_)
    for _ in range(TOP_K):
        mx = jnp.max(key, axis=-1, keepdims=True)
        sel = key == mx
        mask = jnp.logical_or(mask, sel)
        key = jnp.where(sel, -1.0, key)

    cp = jnp.where(mask, p, 0.0)
    combine = cp / jnp.sum(cp, axis=-1, keepdims=True)

    prow = jax.lax.broadcasted_iota(jnp.int32, (E, E * R), 0)
    pcol = jax.lax.broadcasted_iota(jnp.int32, (E, E * R), 1)
    pexp = (jax.lax.shift_right_logical(pcol, 4) == prow).astype(jnp.float32)

    w = jnp.dot(combine, pexp, preferred_element_type=jnp.float32)
    lora = jnp.dot((h * w).astype(jnp.bfloat16), b2_ref[:],
                   preferred_element_type=jnp.float32)
    out_ref[:] = base + bb_ref[:] + lora


@jax.jit
def kernel(x, W_base, b_base, W_router, lora_A, lora_B):
    orig_shape = x.shape
    x_flat = x.reshape(-1, D_IN)
    N = x_flat.shape[0]

    AB3 = jnp.concatenate(
        [lora_A.reshape(E * R, D_IN), W_base, W_router], axis=0
    ).astype(jnp.bfloat16)
    B2 = (SCALING * lora_B.transpose(0, 2, 1).reshape(E * R, D_OUT)
          ).astype(jnp.bfloat16)
    bb = b_base.reshape(1, D_OUT)

    grid = (N // TILE,)
    out = pl.pallas_call(
        _moe_lora_kernel,
        grid=grid,
        in_specs=[
            pl.BlockSpec((TILE, D_IN), lambda i: (i, 0)),
            pl.BlockSpec((E * R + D_OUT + E, D_IN), lambda i: (0, 0)),
            pl.BlockSpec((1, D_OUT), lambda i: (0, 0)),
            pl.BlockSpec((E * R, D_OUT), lambda i: (0, 0)),
        ],
        out_specs=pl.BlockSpec((TILE, D_OUT), lambda i: (i, 0)),
        out_shape=jax.ShapeDtypeStruct((N, D_OUT), x.dtype),
    )(x_flat, AB3, bb, B2)
    return out.reshape(orig_shape[:-1] + (D_OUT,))


# speedup vs baseline: 1.1118x; 1.1118x over previous
import jax
import jax.numpy as jnp
from jax.experimental import pallas as pl

E = 64
TOP_K = 8
R = 16
D_IN = 768
D_OUT = 768
SCALING = 32.0 / R

TILE = 512

_DN_NT = (((1,), (1,)), ((), ()))


def _moe_lora_kernel(x_ref, wb_ref, bb_ref, wr_ref, a2_ref, b2_ref, out_ref):
    xt = x_ref[:]
    xb = xt.astype(jnp.bfloat16)
    logits = jax.lax.dot_general(xt, wr_ref[:], _DN_NT,
                                 preferred_element_type=jnp.float32)
    p = jnp.exp(logits)

    colid = jax.lax.broadcasted_iota(jnp.int32, p.shape, 1)
    pbits = jax.lax.bitcast_convert_type(p, jnp.int32)
    keyi = jnp.bitwise_or(jnp.bitwise_and(pbits, jnp.int32(~63)),
                          (E - 1) - colid)
    key = jax.lax.bitcast_convert_type(keyi, jnp.float32)
    mask = jnp.zeros_like(p, dtype=jnp.bool_)
    for _ in range(TOP_K):
        mx = jnp.max(key, axis=-1, keepdims=True)
        sel = key == mx
        mask = jnp.logical_or(mask, sel)
        key = jnp.where(sel, -1.0, key)

    cp = jnp.where(mask, p, 0.0)
    combine = cp / jnp.sum(cp, axis=-1, keepdims=True)

    prow = jax.lax.broadcasted_iota(jnp.int32, (E, E * R), 0)
    pcol = jax.lax.broadcasted_iota(jnp.int32, (E, E * R), 1)
    pexp = (jax.lax.shift_right_logical(pcol, 4) == prow).astype(jnp.float32)

    h = jax.lax.dot_general(xb, a2_ref[:], _DN_NT,
                            preferred_element_type=jnp.float32)
    w = jnp.dot(combine, pexp, preferred_element_type=jnp.float32)
    lora = jnp.dot((h * w).astype(jnp.bfloat16), b2_ref[:],
                   preferred_element_type=jnp.float32)
    base = jax.lax.dot_general(xb, wb_ref[:], _DN_NT,
                               preferred_element_type=jnp.float32)
    out_ref[:] = base + bb_ref[:] + lora


@jax.jit
def kernel(x, W_base, b_base, W_router, lora_A, lora_B):
    orig_shape = x.shape
    x_flat = x.reshape(-1, D_IN)
    N = x_flat.shape[0]

    A2 = lora_A.reshape(E * R, D_IN).astype(jnp.bfloat16)
    B2 = (SCALING * lora_B.transpose(0, 2, 1).reshape(E * R, D_OUT)
          ).astype(jnp.bfloat16)
    Wb = W_base.astype(jnp.bfloat16)
    bb = b_base.reshape(1, D_OUT)

    grid = (N // TILE,)
    out = pl.pallas_call(
        _moe_lora_kernel,
        grid=grid,
        in_specs=[
            pl.BlockSpec((TILE, D_IN), lambda i: (i, 0)),
            pl.BlockSpec((D_OUT, D_IN), lambda i: (0, 0)),
            pl.BlockSpec((1, D_OUT), lambda i: (0, 0)),
            pl.BlockSpec((E, D_IN), lambda i: (0, 0)),
            pl.BlockSpec((E * R, D_IN), lambda i: (0, 0)),
            pl.BlockSpec((E * R, D_OUT), lambda i: (0, 0)),
        ],
        out_specs=pl.BlockSpec((TILE, D_OUT), lambda i: (i, 0)),
        out_shape=jax.ShapeDtypeStruct((N, D_OUT), x.dtype),
    )(x_flat, Wb, bb, W_router, A2, B2)
    return out.reshape(orig_shape[:-1] + (D_OUT,))
